# Initial kernel scaffold; baseline (speedup 1.0000x reference)
import jax
import jax.numpy as jnp
from jax.experimental import pallas as pl


def kernel(x, edge_index, batch, node_attr, edge_attr, glob_attr, params):
    raise NotImplementedError("write your pallas kernel here")


# speedup vs baseline: 2.2284x; 2.2284x over previous
import jax
import jax.numpy as jnp
from jax import lax
from jax.experimental import pallas as pl
from jax.experimental.pallas import tpu as pltpu
from jax.experimental.pallas import tpu_sc as plsc

N = 10000
E = 320000
NB = 8
NC = 2
NS = 16
NW = NC * NS
K = 128
NBLK = E // K
BPT = -(-NBLK // NW)
ROWS_PT = N // NS

_F32 = jnp.float32


def _dup2(a, w, b):
    h0 = jnp.maximum(a[:, 0:1] * w + b, 0.0)
    h1 = jnp.maximum(a[:, 1:2] * w + b, 0.0)
    return jnp.concatenate([h0, h1], axis=1)


def _dot(a, b):
    return jnp.dot(a, b, preferred_element_type=_F32)


def _tc_node_pre(x_ref, wn_ref, bn_ref, w1a_ref, w1b_ref,
                 ne_ref, a_ref, b_ref):
    ne = _dup2(x_ref[...], wn_ref[...], bn_ref[...])
    ne_ref[...] = ne
    a_ref[...] = _dot(ne, w1a_ref[...])
    b_ref[...] = _dot(ne, w1b_ref[...])


def _tc_edge_pre(sp_ref, w2e_ref, b2e_ref, w1c_ref, be1_ref,
                 w1e_ref, b1e_ref, v1a_ref, be2_ref,
                 c_ref, s_ref, sp16_ref):
    sp = sp_ref[...]
    inv = jnp.where(sp != 0.0, 1.0 / sp, 0.0)
    ie = _dup2(inv, w2e_ref[...], b2e_ref[...])
    c_ref[...] = _dot(ie, w1c_ref[...]) + be1_ref[...]
    se = _dup2(sp, w1e_ref[...], b1e_ref[...])
    s_ref[...] = _dot(se, v1a_ref[...]) + be2_ref[...]
    blk = sp.shape[0]
    sp16_ref[...] = jnp.concatenate(
        [sp, jnp.zeros((blk, 14), _F32)], axis=1)


def _tc_w2(t_ref, w2_ref, b2_ref, o_ref):
    o_ref[...] = _dot(t_ref[...], w2_ref[...]) + b2_ref[...]


def _tc_mid(ebp_ref, ebm_ref, ss_ref,
            w2enc_ref, benc_ref, v1_ref,
            pw1_ref, pb1_ref, pw2_ref, pb2_ref,
            w1o_ref, pp_ref, pm_ref, idsp_ref):
    ebp = ebp_ref[0] + ebp_ref[1]
    ebm = ebm_ref[0] + ebm_ref[1]
    ss = ss_ref[0] + ss_ref[1]
    ssum = ss[:, 0:2]
    inv = jnp.where(ssum != 0.0, 1.0 / ssum, 0.0)
    idsp = _dup2(inv, w2enc_ref[...], benc_ref[...])
    pw1 = pw1_ref[...]
    h = jnp.maximum(_dot(ebp, pw1[0:64]) + _dot(ebm, pw1[64:128])
                    + _dot(idsp, pw1[128:192]) + pb1_ref[...], 0.0)
    w1o_ref[...] = _dot(h, pw2_ref[...]) + pb2_ref[...]
    v1b = v1_ref[...][64:128]
    pp_ref[...] = _dot(ebp, v1b)
    pm_ref[...] = _dot(ebm, v1b)
    idsp_ref[...] = idsp


def _tc_final(ne_ref, w1_ref, vp_ref, vm_ref, idsp_ref, batch_ref, glob_ref,
              wg_ref, bg_ref,
              pw1_ref, pb1_ref, pw2_ref, pb2_ref,
              ew1_ref, eb1_ref, ew2_ref, eb2_ref, dw_ref, db_ref,
              out_ref):
    eb2p = vp_ref[0] + vp_ref[1]
    eb2m = vm_ref[0] + vm_ref[1]
    idsp = idsp_ref[...]
    pw1 = pw1_ref[...]
    h = jnp.maximum(_dot(eb2p, pw1[0:64]) + _dot(eb2m, pw1[64:128])
                    + _dot(idsp, pw1[128:192]) + pb1_ref[...], 0.0)
    w2 = _dot(h, pw2_ref[...]) + pb2_ref[...]
    ge0 = jnp.maximum(glob_ref[...] * wg_ref[...] + bg_ref[...], 0.0)
    ge = jnp.concatenate([ge0, ge0], axis=1)
    bidx = batch_ref[0, 0, :]
    blk = bidx.shape[0]
    oh = (bidx[:, None]
          == lax.broadcasted_iota(jnp.int32, (blk, NB), 1)).astype(_F32)
    gb = _dot(oh, ge)
    ne = ne_ref[...]
    w1v = w1_ref[...]
    ew1 = ew1_ref[...]
    h2 = jnp.maximum(_dot(ne, ew1[0:128]) + _dot(w2, ew1[128:256])
                     + _dot(gb, ew1[256:288]) + _dot(w1v, ew1[288:416])
                     + eb1_ref[...], 0.0)
    res = _dot(h2, ew2_ref[...]) + eb2_ref[...]
    out_ref[...] = _dot(res, dw_ref[...]) + db_ref[...]


def _full(shape):
    nd = len(shape)
    return pl.BlockSpec(shape, lambda i, _nd=nd: (0,) * _nd)


def _rows(blk, width):
    return pl.BlockSpec((blk, width), lambda i: (i, 0))


def _parts(blk, width):
    return pl.BlockSpec((NC, blk, width), lambda i: (0, i, 0))


_MESH = plsc.VectorSubcoreMesh(core_axis_name="c", subcore_axis_name="s",
                               num_cores=NC, num_subcores=NS)
_SC_PARAMS = pltpu.CompilerParams(use_tc_tiling_on_sc=False)


def _sc_gather1(a, b, c, row2d, col2d, t1_o,
                idx_r, idx_c, abuf, bbuf, cbuf, tbuf, sem1, sem2, sem3):
    cc = lax.axis_index("c")
    s = lax.axis_index("s")
    w = s * NC + cc

    def body(i, carry):
        bb = w + i * NW

        @pl.when(bb < NBLK)
        def _():
            pltpu.sync_copy(row2d.at[bb], idx_r)
            pltpu.sync_copy(col2d.at[bb], idx_c)
            d1 = pltpu.async_copy(a.at[idx_r], abuf, sem1)
            d2 = pltpu.async_copy(b.at[idx_c], bbuf, sem2)
            d3 = pltpu.async_copy(c.at[pl.ds(bb * K, K)], cbuf, sem3)
            d1.wait()
            d2.wait()
            d3.wait()

            def compute(e, cy):
                for j in range(8):
                    sl = pl.ds(j * 16, 16)
                    tbuf[e, sl] = jnp.maximum(
                        abuf[e, sl] + bbuf[e, sl] + cbuf[e, sl], 0.0)
                return cy

            lax.fori_loop(0, K, compute, 0)
            pltpu.sync_copy(tbuf, t1_o.at[pl.ds(bb * K, K)])

        return carry

    lax.fori_loop(0, BPT, body, 0)


def _sc_gather2(pp, pm, sarr, row2d, col2d, t2p_o, t2m_o,
                idx_r, idx_c, abuf, bbuf, cbuf, tpbuf, tmbuf,
                sem1, sem2, sem3):
    cc = lax.axis_index("c")
    s = lax.axis_index("s")
    w = s * NC + cc

    def body(i, carry):
        bb = w + i * NW

        @pl.when(bb < NBLK)
        def _():
            pltpu.sync_copy(row2d.at[bb], idx_r)
            pltpu.sync_copy(col2d.at[bb], idx_c)
            d1 = pltpu.async_copy(pp.at[idx_c], abuf, sem1)
            d2 = pltpu.async_copy(pm.at[idx_r], bbuf, sem2)
            d3 = pltpu.async_copy(sarr.at[pl.ds(bb * K, K)], cbuf, sem3)
            d1.wait()
            d2.wait()
            d3.wait()

            def compute(e, cy):
                for j in range(8):
                    sl = pl.ds(j * 16, 16)
                    cv = cbuf[e, sl]
                    tpbuf[e, sl] = jnp.maximum(cv + abuf[e, sl], 0.0)
                    tmbuf[e, sl] = jnp.maximum(cv + bbuf[e, sl], 0.0)
                return cy

            lax.fori_loop(0, K, compute, 0)
            pltpu.sync_copy(tpbuf, t2p_o.at[pl.ds(bb * K, K)])
            pltpu.sync_copy(tmbuf, t2m_o.at[pl.ds(bb * K, K)])

        return carry

    lax.fori_loop(0, BPT, body, 0)


def _sc_scatter1(ep, row2d, col2d, sp16, z64, z16,
                 ebp_o, ebm_o, ss_o,
                 up_sh, um_sh, ss_sh,
                 idx_r, idx_c, ebuf, p16, sem1, sem2):
    cc = lax.axis_index("c")
    s = lax.axis_index("s")
    w = s * NC + cc
    rows = pl.ds(s * ROWS_PT, ROWS_PT)

    pltpu.sync_copy(z64, up_sh.at[rows])
    pltpu.sync_copy(z64, um_sh.at[rows])
    pltpu.sync_copy(z16, ss_sh.at[rows])
    plsc.subcore_barrier()

    def body(i, carry):
        bb = w + i * NW

        @pl.when(bb < NBLK)
        def _():
            pltpu.sync_copy(row2d.at[bb], idx_r)
            pltpu.sync_copy(col2d.at[bb], idx_c)
            d1 = pltpu.async_copy(ep.at[pl.ds(bb * K, K)], ebuf, sem1)
            d2 = pltpu.async_copy(sp16.at[pl.ds(bb * K, K)], p16, sem2)
            d1.wait()
            d2.wait()
            pltpu.sync_copy(ebuf, up_sh.at[idx_r], add=True)
            pltpu.sync_copy(ebuf, um_sh.at[idx_c], add=True)
            pltpu.sync_copy(p16, ss_sh.at[idx_r], add=True)
            pltpu.sync_copy(p16, ss_sh.at[idx_c], add=True)

        return carry

    lax.fori_loop(0, BPT, body, 0)
    plsc.subcore_barrier()
    dst = pl.ds(cc * N + s * ROWS_PT, ROWS_PT)
    pltpu.sync_copy(up_sh.at[rows], ebp_o.at[dst])
    pltpu.sync_copy(um_sh.at[rows], ebm_o.at[dst])
    pltpu.sync_copy(ss_sh.at[rows], ss_o.at[dst])


def _sc_scatter2(epp, epm, row2d, col2d, z64,
                 vp_o, vm_o,
                 vp_sh, vm_sh,
                 idx_r, idx_c, pbuf, mbuf, sem1, sem2):
    cc = lax.axis_index("c")
    s = lax.axis_index("s")
    w = s * NC + cc
    rows = pl.ds(s * ROWS_PT, ROWS_PT)

    pltpu.sync_copy(z64, vp_sh.at[rows])
    pltpu.sync_copy(z64, vm_sh.at[rows])
    plsc.subcore_barrier()

    def body(i, carry):
        bb = w + i * NW

        @pl.when(bb < NBLK)
        def _():
            pltpu.sync_copy(row2d.at[bb], idx_r)
            pltpu.sync_copy(col2d.at[bb], idx_c)
            d1 = pltpu.async_copy(epp.at[pl.ds(bb * K, K)], pbuf, sem1)
            d2 = pltpu.async_copy(epm.at[pl.ds(bb * K, K)], mbuf, sem2)
            d1.wait()
            d2.wait()
            pltpu.sync_copy(pbuf, vp_sh.at[idx_c], add=True)
            pltpu.sync_copy(mbuf, vm_sh.at[idx_r], add=True)

        return carry

    lax.fori_loop(0, BPT, body, 0)
    plsc.subcore_barrier()
    dst = pl.ds(cc * N + s * ROWS_PT, ROWS_PT)
    pltpu.sync_copy(vp_sh.at[rows], vp_o.at[dst])
    pltpu.sync_copy(vm_sh.at[rows], vm_o.at[dst])


def _sds(shape, dtype=_F32):
    return jax.ShapeDtypeStruct(shape, dtype)


_gather1 = pl.kernel(
    _sc_gather1, mesh=_MESH, compiler_params=_SC_PARAMS,
    out_type=[_sds((E, 128))],
    scratch_types=[
        pltpu.VMEM((K,), jnp.int32), pltpu.VMEM((K,), jnp.int32),
        pltpu.VMEM((K, 128), _F32), pltpu.VMEM((K, 128), _F32),
        pltpu.VMEM((K, 128), _F32), pltpu.VMEM((K, 128), _F32),
        pltpu.SemaphoreType.DMA, pltpu.SemaphoreType.DMA,
        pltpu.SemaphoreType.DMA,
    ])

_gather2 = pl.kernel(
    _sc_gather2, mesh=_MESH, compiler_params=_SC_PARAMS,
    out_type=[_sds((E, 128)), _sds((E, 128))],
    scratch_types=[
        pltpu.VMEM((K,), jnp.int32), pltpu.VMEM((K,), jnp.int32),
        pltpu.VMEM((K, 128), _F32), pltpu.VMEM((K, 128), _F32),
        pltpu.VMEM((K, 128), _F32), pltpu.VMEM((K, 128), _F32),
        pltpu.VMEM((K, 128), _F32),
        pltpu.SemaphoreType.DMA, pltpu.SemaphoreType.DMA,
        pltpu.SemaphoreType.DMA,
    ])

_scatter1 = pl.kernel(
    _sc_scatter1, mesh=_MESH, compiler_params=_SC_PARAMS,
    out_type=[_sds((NC * N, 64)), _sds((NC * N, 64)), _sds((NC * N, 16))],
    scratch_types=[
        pltpu.VMEM_SHARED((N, 64), _F32), pltpu.VMEM_SHARED((N, 64), _F32),
        pltpu.VMEM_SHARED((N, 16), _F32),
        pltpu.VMEM((K,), jnp.int32), pltpu.VMEM((K,), jnp.int32),
        pltpu.VMEM((K, 64), _F32), pltpu.VMEM((K, 16), _F32),
        pltpu.SemaphoreType.DMA, pltpu.SemaphoreType.DMA,
    ])

_scatter2 = pl.kernel(
    _sc_scatter2, mesh=_MESH, compiler_params=_SC_PARAMS,
    out_type=[_sds((NC * N, 64)), _sds((NC * N, 64))],
    scratch_types=[
        pltpu.VMEM_SHARED((N, 64), _F32), pltpu.VMEM_SHARED((N, 64), _F32),
        pltpu.VMEM((K,), jnp.int32), pltpu.VMEM((K,), jnp.int32),
        pltpu.VMEM((K, 64), _F32), pltpu.VMEM((K, 64), _F32),
        pltpu.SemaphoreType.DMA, pltpu.SemaphoreType.DMA,
    ])


def _impl(x, edge_index, batch, edge_attr, glob_attr, p):
    r2 = lambda a: a.reshape(1, -1)
    w1 = p['phi_e1_W1']

    bn1, nbk = 2000, N // 2000
    ne, amat, bmat = pl.pallas_call(
        _tc_node_pre,
        grid=(nbk,),
        in_specs=[_rows(bn1, 2), _full((1, 64)), _full((1, 64)),
                  _full((128, 128)), _full((128, 128))],
        out_specs=[_rows(bn1, 128), _rows(bn1, 128), _rows(bn1, 128)],
        out_shape=[_sds((N, 128)), _sds((N, 128)), _sds((N, 128))],
    )(x, r2(p['enc1_node_W']), r2(p['enc1_node_b']),
      w1[0:128], w1[128:256])

    be, ebk = 2000, E // 2000
    cmat, smat, sp16 = pl.pallas_call(
        _tc_edge_pre,
        grid=(ebk,),
        in_specs=[_rows(be, 2), _full((1, 32)), _full((1, 32)),
                  _full((64, 128)), _full((1, 128)),
                  _full((1, 32)), _full((1, 32)),
                  _full((64, 128)), _full((1, 128))],
        out_specs=[_rows(be, 128), _rows(be, 128), _rows(be, 16)],
        out_shape=[_sds((E, 128)), _sds((E, 128)), _sds((E, 16))],
    )(edge_attr, r2(p['enc2_edge_W']), r2(p['enc2_edge_b']),
      w1[256:320], r2(p['phi_e1_b1']),
      r2(p['enc1_edge_W']), r2(p['enc1_edge_b']),
      p['phi_e2_W1'][0:64], r2(p['phi_e2_b1']))

    row2d = edge_index[0].reshape(NBLK, K)
    col2d = edge_index[1].reshape(NBLK, K)
    z64 = jnp.zeros((ROWS_PT, 64), _F32)
    z16 = jnp.zeros((ROWS_PT, 16), _F32)

    def w2call(t, w2, b2):
        return pl.pallas_call(
            _tc_w2, grid=(ebk,),
            in_specs=[_rows(be, 128), _full((128, 64)), _full((1, 64))],
            out_specs=_rows(be, 64),
            out_shape=_sds((E, 64)),
        )(t, w2, r2(b2))

    t1, = _gather1(amat, bmat, cmat, row2d, col2d)
    ep1 = w2call(t1, p['phi_e1_W2'], p['phi_e1_b2'])
    ebp_p, ebm_p, ss_p = _scatter1(ep1, row2d, col2d, sp16, z64, z16)

    bm, mbk = 2000, N // 2000
    w1v, pp, pm, idsp = pl.pallas_call(
        _tc_mid,
        grid=(mbk,),
        in_specs=[_parts(bm, 64)] * 2 + [_parts(bm, 16)] +
                 [_full((1, 32)), _full((1, 32)), _full((128, 128)),
                  _full((192, 128)), _full((1, 128)),
                  _full((128, 128)), _full((1, 128))],
        out_specs=[_rows(bm, 128), _rows(bm, 128), _rows(bm, 128),
                   _rows(bm, 64)],
        out_shape=[_sds((N, 128))] * 3 + [_sds((N, 64))],
    )(ebp_p.reshape(NC, N, 64), ebm_p.reshape(NC, N, 64),
      ss_p.reshape(NC, N, 16),
      r2(p['enc2_edge_W']), r2(p['enc2_edge_b']), p['phi_e2_W1'],
      p['phi_v1_W1'], r2(p['phi_v1_b1']),
      p['phi_v1_W2'], r2(p['phi_v1_b2']))

    t2p, t2m = _gather2(pp, pm, smat, row2d, col2d)
    ep2p = w2call(t2p, p['phi_e2_W2'], p['phi_e2_b2'])
    ep2m = w2call(t2m, p['phi_e2_W2'], p['phi_e2_b2'])
    vp_p, vm_p = _scatter2(ep2p, ep2m, row2d, col2d, z64)

    out = pl.pallas_call(
        _tc_final,
        grid=(mbk,),
        in_specs=[_rows(bm, 128), _rows(bm, 128)] +
                 [_parts(bm, 64)] * 2 +
                 [_rows(bm, 64),
                  pl.BlockSpec((1, 1, bm), lambda i: (i, 0, 0)),
                  _full((NB, 1)),
                  _full((1, 16)), _full((1, 16)),
                  _full((192, 128)), _full((1, 128)),
                  _full((128, 128)), _full((1, 128)),
                  _full((416, 128)), _full((1, 128)),
                  _full((128, 128)), _full((1, 128)),
                  _full((128, 2)), _full((1, 2))],
        out_specs=_rows(bm, 2),
        out_shape=_sds((N, 2)),
    )(ne, w1v,
      vp_p.reshape(NC, N, 64), vm_p.reshape(NC, N, 64),
      idsp, batch.reshape(mbk, 1, bm), glob_attr,
      r2(p['enc1_glob_W']), r2(p['enc1_glob_b']),
      p['phi_v2_W1'], r2(p['phi_v2_b1']),
      p['phi_v2_W2'], r2(p['phi_v2_b2']),
      p['ext_dec_W1'], r2(p['ext_dec_b1']),
      p['ext_dec_W2'], r2(p['ext_dec_b2']),
      p['dec_W'], r2(p['dec_b']))
    return out


_run = jax.jit(_impl)


def kernel(x, edge_index, batch, node_attr, edge_attr, glob_attr, params):
    del node_attr
    return _run(x, edge_index, batch, edge_attr, glob_attr, params)


# speedup vs baseline: 2.3241x; 1.0429x over previous
import jax
import jax.numpy as jnp
from jax import lax
from jax.experimental import pallas as pl
from jax.experimental.pallas import tpu as pltpu
from jax.experimental.pallas import tpu_sc as plsc

N = 10000
E = 320000
NB = 8
NC = 2
NS = 16
NW = NC * NS
K = 128
NBLK = E // K
BPT = -(-NBLK // NW)
ROWS_PT = N // NS

_F32 = jnp.float32


def _dup2(a, w, b):
    h0 = jnp.maximum(a[:, 0:1] * w + b, 0.0)
    h1 = jnp.maximum(a[:, 1:2] * w + b, 0.0)
    return jnp.concatenate([h0, h1], axis=1)


def _dot(a, b):
    return jnp.dot(a, b, preferred_element_type=_F32)


def _tc_node_pre(x_ref, wn_ref, bn_ref, w1a_ref, w1b_ref,
                 ne_ref, a_ref, b_ref):
    ne = _dup2(x_ref[...], wn_ref[...], bn_ref[...])
    ne_ref[...] = ne
    a_ref[...] = _dot(ne, w1a_ref[...])
    b_ref[...] = _dot(ne, w1b_ref[...])


def _tc_edge_pre(sp_ref, w2e_ref, b2e_ref, w1c_ref, be1_ref,
                 w1e_ref, b1e_ref, v1a_ref, be2_ref,
                 c_ref, s_ref, sp16_ref):
    sp = sp_ref[...]
    inv = jnp.where(sp != 0.0, 1.0 / sp, 0.0)
    ie = _dup2(inv, w2e_ref[...], b2e_ref[...])
    c_ref[...] = _dot(ie, w1c_ref[...]) + be1_ref[...]
    se = _dup2(sp, w1e_ref[...], b1e_ref[...])
    s_ref[...] = _dot(se, v1a_ref[...]) + be2_ref[...]
    blk = sp.shape[0]
    sp16_ref[...] = jnp.concatenate(
        [sp, jnp.zeros((blk, 14), _F32)], axis=1)


def _tc_w2(t_ref, w2_ref, b2_ref, o_ref):
    o_ref[...] = _dot(t_ref[...], w2_ref[...]) + b2_ref[...]


def _tc_mid(ebp_ref, ebm_ref, ss_ref,
            w2enc_ref, benc_ref, v1_ref,
            pw1_ref, pb1_ref, pw2_ref, pb2_ref,
            w1o_ref, pp_ref, pm_ref, idsp_ref):
    ebp = ebp_ref[0] + ebp_ref[1]
    ebm = ebm_ref[0] + ebm_ref[1]
    ss = ss_ref[0] + ss_ref[1]
    ssum = ss[:, 0:2]
    inv = jnp.where(ssum != 0.0, 1.0 / ssum, 0.0)
    idsp = _dup2(inv, w2enc_ref[...], benc_ref[...])
    pw1 = pw1_ref[...]
    h = jnp.maximum(_dot(ebp, pw1[0:64]) + _dot(ebm, pw1[64:128])
                    + _dot(idsp, pw1[128:192]) + pb1_ref[...], 0.0)
    w1o_ref[...] = _dot(h, pw2_ref[...]) + pb2_ref[...]
    v1b = v1_ref[...][64:128]
    pp_ref[...] = _dot(ebp, v1b)
    pm_ref[...] = _dot(ebm, v1b)
    idsp_ref[...] = idsp


def _tc_final(ne_ref, w1_ref, vp_ref, vm_ref, idsp_ref, batch_ref, glob_ref,
              wg_ref, bg_ref,
              pw1_ref, pb1_ref, pw2_ref, pb2_ref,
              ew1_ref, eb1_ref, ew2_ref, eb2_ref, dw_ref, db_ref,
              out_ref):
    eb2p = vp_ref[0] + vp_ref[1]
    eb2m = vm_ref[0] + vm_ref[1]
    idsp = idsp_ref[...]
    pw1 = pw1_ref[...]
    h = jnp.maximum(_dot(eb2p, pw1[0:64]) + _dot(eb2m, pw1[64:128])
                    + _dot(idsp, pw1[128:192]) + pb1_ref[...], 0.0)
    w2 = _dot(h, pw2_ref[...]) + pb2_ref[...]
    ge0 = jnp.maximum(glob_ref[...] * wg_ref[...] + bg_ref[...], 0.0)
    ge = jnp.concatenate([ge0, ge0], axis=1)
    bidx = batch_ref[0, 0, :]
    blk = bidx.shape[0]
    oh = (bidx[:, None]
          == lax.broadcasted_iota(jnp.int32, (blk, NB), 1)).astype(_F32)
    gb = _dot(oh, ge)
    ne = ne_ref[...]
    w1v = w1_ref[...]
    ew1 = ew1_ref[...]
    h2 = jnp.maximum(_dot(ne, ew1[0:128]) + _dot(w2, ew1[128:256])
                     + _dot(gb, ew1[256:288]) + _dot(w1v, ew1[288:416])
                     + eb1_ref[...], 0.0)
    res = _dot(h2, ew2_ref[...]) + eb2_ref[...]
    out_ref[...] = _dot(res, dw_ref[...]) + db_ref[...]


def _full(shape):
    nd = len(shape)
    return pl.BlockSpec(shape, lambda i, _nd=nd: (0,) * _nd)


def _rows(blk, width):
    return pl.BlockSpec((blk, width), lambda i: (i, 0))


def _parts(blk, width):
    return pl.BlockSpec((NC, blk, width), lambda i: (0, i, 0))


_MESH = plsc.VectorSubcoreMesh(core_axis_name="c", subcore_axis_name="s",
                               num_cores=NC, num_subcores=NS)
_SC_PARAMS = pltpu.CompilerParams(use_tc_tiling_on_sc=False)


def _sc_gather1(a, b, c, row2d, col2d, t1_o,
                idx_r0, idx_c0, abuf0, bbuf0, cbuf0,
                idx_r1, idx_c1, abuf1, bbuf1, cbuf1, tbuf,
                sem10, sem20, sem30, sem11, sem21, sem31):
    cc = lax.axis_index("c")
    s = lax.axis_index("s")
    w = s * NC + cc
    slots = ((idx_r0, idx_c0, abuf0, bbuf0, cbuf0, sem10, sem20, sem30),
             (idx_r1, idx_c1, abuf1, bbuf1, cbuf1, sem11, sem21, sem31))

    def body(i, carry):
        descs = []
        for q in range(2):
            bb = w + (2 * i + q) * NW
            ir, ic, ab, bbf, cb, s1, s2, s3 = slots[q]

            @pl.when(bb < NBLK)
            def _(bb=bb, ir=ir, ic=ic, ab=ab, bbf=bbf, cb=cb,
                  s1=s1, s2=s2, s3=s3):
                pltpu.sync_copy(row2d.at[bb], ir)
                pltpu.sync_copy(col2d.at[bb], ic)
                pltpu.async_copy(a.at[ir], ab, s1)
                pltpu.async_copy(b.at[ic], bbf, s2)
                pltpu.async_copy(c.at[pl.ds(bb * K, K)], cb, s3)

        for q in range(2):
            bb = w + (2 * i + q) * NW
            ir, ic, ab, bbf, cb, s1, s2, s3 = slots[q]

            @pl.when(bb < NBLK)
            def _(bb=bb, ab=ab, bbf=bbf, cb=cb, s1=s1, s2=s2, s3=s3):
                pltpu.make_async_copy(a, ab, s1).wait()
                pltpu.make_async_copy(a, bbf, s2).wait()
                pltpu.make_async_copy(c.at[pl.ds(bb * K, K)], cb, s3).wait()

                def compute(e, cy):
                    for j in range(8):
                        sl = pl.ds(j * 16, 16)
                        tbuf[e, sl] = jnp.maximum(
                            ab[e, sl] + bbf[e, sl] + cb[e, sl], 0.0)
                    return cy

                lax.fori_loop(0, K, compute, 0)
                pltpu.sync_copy(tbuf, t1_o.at[pl.ds(bb * K, K)])

        return carry

    lax.fori_loop(0, (BPT + 1) // 2, body, 0)


def _sc_gather2(pp, pm, sarr, row2d, col2d, t2p_o, t2m_o,
                idx_r, idx_c, abuf, bbuf, cbuf, tpbuf, tmbuf,
                sem1, sem2, sem3):
    cc = lax.axis_index("c")
    s = lax.axis_index("s")
    w = s * NC + cc

    def body(i, carry):
        bb = w + i * NW

        @pl.when(bb < NBLK)
        def _():
            pltpu.sync_copy(row2d.at[bb], idx_r)
            pltpu.sync_copy(col2d.at[bb], idx_c)
            d1 = pltpu.async_copy(pp.at[idx_c], abuf, sem1)
            d2 = pltpu.async_copy(pm.at[idx_r], bbuf, sem2)
            d3 = pltpu.async_copy(sarr.at[pl.ds(bb * K, K)], cbuf, sem3)
            d1.wait()
            d2.wait()
            d3.wait()

            def compute(e, cy):
                for j in range(8):
                    sl = pl.ds(j * 16, 16)
                    cv = cbuf[e, sl]
                    tpbuf[e, sl] = jnp.maximum(cv + abuf[e, sl], 0.0)
                    tmbuf[e, sl] = jnp.maximum(cv + bbuf[e, sl], 0.0)
                return cy

            lax.fori_loop(0, K, compute, 0)
            pltpu.sync_copy(tpbuf, t2p_o.at[pl.ds(bb * K, K)])
            pltpu.sync_copy(tmbuf, t2m_o.at[pl.ds(bb * K, K)])

        return carry

    lax.fori_loop(0, BPT, body, 0)


def _sc_scatter1(ep, row2d, col2d, sp16, z64, z16,
                 ebp_o, ebm_o, ss_o,
                 up_sh, um_sh, ss_sh,
                 idx_r, idx_c, ebuf, p16, sem1, sem2):
    cc = lax.axis_index("c")
    s = lax.axis_index("s")
    w = s * NC + cc
    rows = pl.ds(s * ROWS_PT, ROWS_PT)

    pltpu.sync_copy(z64, up_sh.at[rows])
    pltpu.sync_copy(z64, um_sh.at[rows])
    pltpu.sync_copy(z16, ss_sh.at[rows])
    plsc.subcore_barrier()

    def body(i, carry):
        bb = w + i * NW

        @pl.when(bb < NBLK)
        def _():
            pltpu.sync_copy(row2d.at[bb], idx_r)
            pltpu.sync_copy(col2d.at[bb], idx_c)
            d1 = pltpu.async_copy(ep.at[pl.ds(bb * K, K)], ebuf, sem1)
            d2 = pltpu.async_copy(sp16.at[pl.ds(bb * K, K)], p16, sem2)
            d1.wait()
            d2.wait()
            pltpu.sync_copy(ebuf, up_sh.at[idx_r], add=True)
            pltpu.sync_copy(ebuf, um_sh.at[idx_c], add=True)
            pltpu.sync_copy(p16, ss_sh.at[idx_r], add=True)
            pltpu.sync_copy(p16, ss_sh.at[idx_c], add=True)

        return carry

    lax.fori_loop(0, BPT, body, 0)
    plsc.subcore_barrier()
    dst = pl.ds(cc * N + s * ROWS_PT, ROWS_PT)
    pltpu.sync_copy(up_sh.at[rows], ebp_o.at[dst])
    pltpu.sync_copy(um_sh.at[rows], ebm_o.at[dst])
    pltpu.sync_copy(ss_sh.at[rows], ss_o.at[dst])


def _sc_scatter2(epp, epm, row2d, col2d, z64,
                 vp_o, vm_o,
                 vp_sh, vm_sh,
                 idx_r, idx_c, pbuf, mbuf, sem1, sem2):
    cc = lax.axis_index("c")
    s = lax.axis_index("s")
    w = s * NC + cc
    rows = pl.ds(s * ROWS_PT, ROWS_PT)

    pltpu.sync_copy(z64, vp_sh.at[rows])
    pltpu.sync_copy(z64, vm_sh.at[rows])
    plsc.subcore_barrier()

    def body(i, carry):
        bb = w + i * NW

        @pl.when(bb < NBLK)
        def _():
            pltpu.sync_copy(row2d.at[bb], idx_r)
            pltpu.sync_copy(col2d.at[bb], idx_c)
            d1 = pltpu.async_copy(epp.at[pl.ds(bb * K, K)], pbuf, sem1)
            d2 = pltpu.async_copy(epm.at[pl.ds(bb * K, K)], mbuf, sem2)
            d1.wait()
            d2.wait()
            pltpu.sync_copy(pbuf, vp_sh.at[idx_c], add=True)
            pltpu.sync_copy(mbuf, vm_sh.at[idx_r], add=True)

        return carry

    lax.fori_loop(0, BPT, body, 0)
    plsc.subcore_barrier()
    dst = pl.ds(cc * N + s * ROWS_PT, ROWS_PT)
    pltpu.sync_copy(vp_sh.at[rows], vp_o.at[dst])
    pltpu.sync_copy(vm_sh.at[rows], vm_o.at[dst])


def _sds(shape, dtype=_F32):
    return jax.ShapeDtypeStruct(shape, dtype)


_gather1 = pl.kernel(
    _sc_gather1, mesh=_MESH, compiler_params=_SC_PARAMS,
    out_type=[_sds((E, 128))],
    scratch_types=[
        pltpu.VMEM((K,), jnp.int32), pltpu.VMEM((K,), jnp.int32),
        pltpu.VMEM((K, 128), _F32), pltpu.VMEM((K, 128), _F32),
        pltpu.VMEM((K, 128), _F32),
        pltpu.VMEM((K,), jnp.int32), pltpu.VMEM((K,), jnp.int32),
        pltpu.VMEM((K, 128), _F32), pltpu.VMEM((K, 128), _F32),
        pltpu.VMEM((K, 128), _F32),
        pltpu.VMEM((K, 128), _F32),
        pltpu.SemaphoreType.DMA, pltpu.SemaphoreType.DMA,
        pltpu.SemaphoreType.DMA, pltpu.SemaphoreType.DMA,
        pltpu.SemaphoreType.DMA, pltpu.SemaphoreType.DMA,
    ])

_gather2 = pl.kernel(
    _sc_gather2, mesh=_MESH, compiler_params=_SC_PARAMS,
    out_type=[_sds((E, 128)), _sds((E, 128))],
    scratch_types=[
        pltpu.VMEM((K,), jnp.int32), pltpu.VMEM((K,), jnp.int32),
        pltpu.VMEM((K, 128), _F32), pltpu.VMEM((K, 128), _F32),
        pltpu.VMEM((K, 128), _F32), pltpu.VMEM((K, 128), _F32),
        pltpu.VMEM((K, 128), _F32),
        pltpu.SemaphoreType.DMA, pltpu.SemaphoreType.DMA,
        pltpu.SemaphoreType.DMA,
    ])

_scatter1 = pl.kernel(
    _sc_scatter1, mesh=_MESH, compiler_params=_SC_PARAMS,
    out_type=[_sds((NC * N, 64)), _sds((NC * N, 64)), _sds((NC * N, 16))],
    scratch_types=[
        pltpu.VMEM_SHARED((N, 64), _F32), pltpu.VMEM_SHARED((N, 64), _F32),
        pltpu.VMEM_SHARED((N, 16), _F32),
        pltpu.VMEM((K,), jnp.int32), pltpu.VMEM((K,), jnp.int32),
        pltpu.VMEM((K, 64), _F32), pltpu.VMEM((K, 16), _F32),
        pltpu.SemaphoreType.DMA, pltpu.SemaphoreType.DMA,
    ])

_scatter2 = pl.kernel(
    _sc_scatter2, mesh=_MESH, compiler_params=_SC_PARAMS,
    out_type=[_sds((NC * N, 64)), _sds((NC * N, 64))],
    scratch_types=[
        pltpu.VMEM_SHARED((N, 64), _F32), pltpu.VMEM_SHARED((N, 64), _F32),
        pltpu.VMEM((K,), jnp.int32), pltpu.VMEM((K,), jnp.int32),
        pltpu.VMEM((K, 64), _F32), pltpu.VMEM((K, 64), _F32),
        pltpu.SemaphoreType.DMA, pltpu.SemaphoreType.DMA,
    ])


def _impl(x, edge_index, batch, edge_attr, glob_attr, p):
    r2 = lambda a: a.reshape(1, -1)
    w1 = p['phi_e1_W1']

    bn1, nbk = 2000, N // 2000
    ne, amat, bmat = pl.pallas_call(
        _tc_node_pre,
        grid=(nbk,),
        in_specs=[_rows(bn1, 2), _full((1, 64)), _full((1, 64)),
                  _full((128, 128)), _full((128, 128))],
        out_specs=[_rows(bn1, 128), _rows(bn1, 128), _rows(bn1, 128)],
        out_shape=[_sds((N, 128)), _sds((N, 128)), _sds((N, 128))],
    )(x, r2(p['enc1_node_W']), r2(p['enc1_node_b']),
      w1[0:128], w1[128:256])

    be, ebk = 2000, E // 2000
    cmat, smat, sp16 = pl.pallas_call(
        _tc_edge_pre,
        grid=(ebk,),
        in_specs=[_rows(be, 2), _full((1, 32)), _full((1, 32)),
                  _full((64, 128)), _full((1, 128)),
                  _full((1, 32)), _full((1, 32)),
                  _full((64, 128)), _full((1, 128))],
        out_specs=[_rows(be, 128), _rows(be, 128), _rows(be, 16)],
        out_shape=[_sds((E, 128)), _sds((E, 128)), _sds((E, 16))],
    )(edge_attr, r2(p['enc2_edge_W']), r2(p['enc2_edge_b']),
      w1[256:320], r2(p['phi_e1_b1']),
      r2(p['enc1_edge_W']), r2(p['enc1_edge_b']),
      p['phi_e2_W1'][0:64], r2(p['phi_e2_b1']))

    row2d = edge_index[0].reshape(NBLK, K)
    col2d = edge_index[1].reshape(NBLK, K)
    z64 = jnp.zeros((ROWS_PT, 64), _F32)
    z16 = jnp.zeros((ROWS_PT, 16), _F32)

    def w2call(t, w2, b2):
        return pl.pallas_call(
            _tc_w2, grid=(ebk,),
            in_specs=[_rows(be, 128), _full((128, 64)), _full((1, 64))],
            out_specs=_rows(be, 64),
            out_shape=_sds((E, 64)),
        )(t, w2, r2(b2))

    t1, = _gather1(amat, bmat, cmat, row2d, col2d)
    ep1 = w2call(t1, p['phi_e1_W2'], p['phi_e1_b2'])
    ebp_p, ebm_p, ss_p = _scatter1(ep1, row2d, col2d, sp16, z64, z16)

    bm, mbk = 2000, N // 2000
    w1v, pp, pm, idsp = pl.pallas_call(
        _tc_mid,
        grid=(mbk,),
        in_specs=[_parts(bm, 64)] * 2 + [_parts(bm, 16)] +
                 [_full((1, 32)), _full((1, 32)), _full((128, 128)),
                  _full((192, 128)), _full((1, 128)),
                  _full((128, 128)), _full((1, 128))],
        out_specs=[_rows(bm, 128), _rows(bm, 128), _rows(bm, 128),
                   _rows(bm, 64)],
        out_shape=[_sds((N, 128))] * 3 + [_sds((N, 64))],
    )(ebp_p.reshape(NC, N, 64), ebm_p.reshape(NC, N, 64),
      ss_p.reshape(NC, N, 16),
      r2(p['enc2_edge_W']), r2(p['enc2_edge_b']), p['phi_e2_W1'],
      p['phi_v1_W1'], r2(p['phi_v1_b1']),
      p['phi_v1_W2'], r2(p['phi_v1_b2']))

    t2p, t2m = _gather2(pp, pm, smat, row2d, col2d)
    ep2p = w2call(t2p, p['phi_e2_W2'], p['phi_e2_b2'])
    ep2m = w2call(t2m, p['phi_e2_W2'], p['phi_e2_b2'])
    vp_p, vm_p = _scatter2(ep2p, ep2m, row2d, col2d, z64)

    out = pl.pallas_call(
        _tc_final,
        grid=(mbk,),
        in_specs=[_rows(bm, 128), _rows(bm, 128)] +
                 [_parts(bm, 64)] * 2 +
                 [_rows(bm, 64),
                  pl.BlockSpec((1, 1, bm), lambda i: (i, 0, 0)),
                  _full((NB, 1)),
                  _full((1, 16)), _full((1, 16)),
                  _full((192, 128)), _full((1, 128)),
                  _full((128, 128)), _full((1, 128)),
                  _full((416, 128)), _full((1, 128)),
                  _full((128, 128)), _full((1, 128)),
                  _full((128, 2)), _full((1, 2))],
        out_specs=_rows(bm, 2),
        out_shape=_sds((N, 2)),
    )(ne, w1v,
      vp_p.reshape(NC, N, 64), vm_p.reshape(NC, N, 64),
      idsp, batch.reshape(mbk, 1, bm), glob_attr,
      r2(p['enc1_glob_W']), r2(p['enc1_glob_b']),
      p['phi_v2_W1'], r2(p['phi_v2_b1']),
      p['phi_v2_W2'], r2(p['phi_v2_b2']),
      p['ext_dec_W1'], r2(p['ext_dec_b1']),
      p['ext_dec_W2'], r2(p['ext_dec_b2']),
      p['dec_W'], r2(p['dec_b']))
    return out


_run = jax.jit(_impl)


def kernel(x, edge_index, batch, node_attr, edge_attr, glob_attr, params):
    del node_attr
    return _run(x, edge_index, batch, edge_attr, glob_attr, params)


# speedup vs baseline: 2.3519x; 1.0120x over previous
import jax
import jax.numpy as jnp
from jax import lax
from jax.experimental import pallas as pl
from jax.experimental.pallas import tpu as pltpu
from jax.experimental.pallas import tpu_sc as plsc

N = 10000
E = 320000
NB = 8
NC = 2
NS = 16
NW = NC * NS
K = 128
NBLK = E // K
BPT = -(-NBLK // NW)
ROWS_PT = N // NS

_F32 = jnp.float32


def _dup2(a, w, b):
    h0 = jnp.maximum(a[:, 0:1] * w + b, 0.0)
    h1 = jnp.maximum(a[:, 1:2] * w + b, 0.0)
    return jnp.concatenate([h0, h1], axis=1)


def _dot(a, b):
    return jnp.dot(a, b, preferred_element_type=_F32)


def _tc_node_pre(x_ref, wn_ref, bn_ref, w1a_ref, w1b_ref,
                 ne_ref, a_ref, b_ref):
    ne = _dup2(x_ref[...], wn_ref[...], bn_ref[...])
    ne_ref[...] = ne
    a_ref[...] = _dot(ne, w1a_ref[...])
    b_ref[...] = _dot(ne, w1b_ref[...])


def _tc_edge_pre(sp_ref, w2e_ref, b2e_ref, w1c_ref, be1_ref,
                 w1e_ref, b1e_ref, v1a_ref, be2_ref,
                 c_ref, s_ref, sp16_ref):
    sp = sp_ref[...]
    inv = jnp.where(sp != 0.0, 1.0 / sp, 0.0)
    ie = _dup2(inv, w2e_ref[...], b2e_ref[...])
    c_ref[...] = _dot(ie, w1c_ref[...]) + be1_ref[...]
    se = _dup2(sp, w1e_ref[...], b1e_ref[...])
    s_ref[...] = _dot(se, v1a_ref[...]) + be2_ref[...]
    blk = sp.shape[0]
    sp16_ref[...] = jnp.concatenate(
        [sp, jnp.zeros((blk, 14), _F32)], axis=1)


def _tc_w2(t_ref, w2_ref, b2_ref, o_ref):
    o_ref[...] = _dot(t_ref[...], w2_ref[...]) + b2_ref[...]


def _tc_mid(ebp_ref, ebm_ref, ss_ref,
            w2enc_ref, benc_ref, v1_ref,
            pw1_ref, pb1_ref, pw2_ref, pb2_ref,
            w1o_ref, pp_ref, pm_ref, idsp_ref):
    ebp = ebp_ref[0] + ebp_ref[1]
    ebm = ebm_ref[0] + ebm_ref[1]
    ss = ss_ref[0] + ss_ref[1]
    ssum = ss[:, 0:2]
    inv = jnp.where(ssum != 0.0, 1.0 / ssum, 0.0)
    idsp = _dup2(inv, w2enc_ref[...], benc_ref[...])
    pw1 = pw1_ref[...]
    h = jnp.maximum(_dot(ebp, pw1[0:64]) + _dot(ebm, pw1[64:128])
                    + _dot(idsp, pw1[128:192]) + pb1_ref[...], 0.0)
    w1o_ref[...] = _dot(h, pw2_ref[...]) + pb2_ref[...]
    v1b = v1_ref[...][64:128]
    pp_ref[...] = _dot(ebp, v1b)
    pm_ref[...] = _dot(ebm, v1b)
    idsp_ref[...] = idsp


def _tc_final(ne_ref, w1_ref, vp_ref, vm_ref, idsp_ref, batch_ref, glob_ref,
              wg_ref, bg_ref,
              pw1_ref, pb1_ref, pw2_ref, pb2_ref,
              ew1_ref, eb1_ref, ew2_ref, eb2_ref, dw_ref, db_ref,
              out_ref):
    eb2p = vp_ref[0] + vp_ref[1]
    eb2m = vm_ref[0] + vm_ref[1]
    idsp = idsp_ref[...]
    pw1 = pw1_ref[...]
    h = jnp.maximum(_dot(eb2p, pw1[0:64]) + _dot(eb2m, pw1[64:128])
                    + _dot(idsp, pw1[128:192]) + pb1_ref[...], 0.0)
    w2 = _dot(h, pw2_ref[...]) + pb2_ref[...]
    ge0 = jnp.maximum(glob_ref[...] * wg_ref[...] + bg_ref[...], 0.0)
    ge = jnp.concatenate([ge0, ge0], axis=1)
    bidx = batch_ref[0, 0, :]
    blk = bidx.shape[0]
    oh = (bidx[:, None]
          == lax.broadcasted_iota(jnp.int32, (blk, NB), 1)).astype(_F32)
    gb = _dot(oh, ge)
    ne = ne_ref[...]
    w1v = w1_ref[...]
    ew1 = ew1_ref[...]
    h2 = jnp.maximum(_dot(ne, ew1[0:128]) + _dot(w2, ew1[128:256])
                     + _dot(gb, ew1[256:288]) + _dot(w1v, ew1[288:416])
                     + eb1_ref[...], 0.0)
    res = _dot(h2, ew2_ref[...]) + eb2_ref[...]
    out_ref[...] = _dot(res, dw_ref[...]) + db_ref[...]


def _full(shape):
    nd = len(shape)
    return pl.BlockSpec(shape, lambda i, _nd=nd: (0,) * _nd)


def _rows(blk, width):
    return pl.BlockSpec((blk, width), lambda i: (i, 0))


def _parts(blk, width):
    return pl.BlockSpec((NC, blk, width), lambda i: (0, i, 0))


_MESH = plsc.VectorSubcoreMesh(core_axis_name="c", subcore_axis_name="s",
                               num_cores=NC, num_subcores=NS)
_SC_PARAMS = pltpu.CompilerParams(use_tc_tiling_on_sc=False)


def _sc_gather1(a, b, c, row2d, col2d, t1_o,
                idx_r0, idx_c0, abuf0, bbuf0, cbuf0,
                idx_r1, idx_c1, abuf1, bbuf1, cbuf1, tbuf,
                sem10, sem20, sem30, sem11, sem21, sem31):
    cc = lax.axis_index("c")
    s = lax.axis_index("s")
    w = s * NC + cc
    slots = ((idx_r0, idx_c0, abuf0, bbuf0, cbuf0, sem10, sem20, sem30),
             (idx_r1, idx_c1, abuf1, bbuf1, cbuf1, sem11, sem21, sem31))

    def body(i, carry):
        descs = []
        for q in range(2):
            bb = w + (2 * i + q) * NW
            ir, ic, ab, bbf, cb, s1, s2, s3 = slots[q]

            @pl.when(bb < NBLK)
            def _(bb=bb, ir=ir, ic=ic, ab=ab, bbf=bbf, cb=cb,
                  s1=s1, s2=s2, s3=s3):
                pltpu.sync_copy(row2d.at[bb], ir)
                pltpu.sync_copy(col2d.at[bb], ic)
                pltpu.async_copy(a.at[ir], ab, s1)
                pltpu.async_copy(b.at[ic], bbf, s2)
                pltpu.async_copy(c.at[pl.ds(bb * K, K)], cb, s3)

        for q in range(2):
            bb = w + (2 * i + q) * NW
            ir, ic, ab, bbf, cb, s1, s2, s3 = slots[q]

            @pl.when(bb < NBLK)
            def _(bb=bb, ab=ab, bbf=bbf, cb=cb, s1=s1, s2=s2, s3=s3):
                pltpu.make_async_copy(a, ab, s1).wait()
                pltpu.make_async_copy(a, bbf, s2).wait()
                pltpu.make_async_copy(c.at[pl.ds(bb * K, K)], cb, s3).wait()

                def compute(e, cy):
                    for j in range(8):
                        sl = pl.ds(j * 16, 16)
                        tbuf[e, sl] = jnp.maximum(
                            ab[e, sl] + bbf[e, sl] + cb[e, sl], 0.0)
                    return cy

                lax.fori_loop(0, K, compute, 0)
                pltpu.sync_copy(tbuf, t1_o.at[pl.ds(bb * K, K)])

        return carry

    lax.fori_loop(0, (BPT + 1) // 2, body, 0)


def _sc_gather2(pp, pm, sarr, row2d, col2d, t2p_o, t2m_o,
                idx_r0, idx_c0, abuf0, bbuf0,
                idx_r1, idx_c1, abuf1, bbuf1, cbuf, tpbuf, tmbuf,
                sem10, sem20, sem11, sem21, sem3):
    cc = lax.axis_index("c")
    s = lax.axis_index("s")
    w = s * NC + cc
    slots = ((idx_r0, idx_c0, abuf0, bbuf0, sem10, sem20),
             (idx_r1, idx_c1, abuf1, bbuf1, sem11, sem21))

    def body(i, carry):
        for q in range(2):
            bb = w + (2 * i + q) * NW
            ir, ic, ab, bbf, s1, s2 = slots[q]

            @pl.when(bb < NBLK)
            def _(bb=bb, ir=ir, ic=ic, ab=ab, bbf=bbf, s1=s1, s2=s2):
                pltpu.sync_copy(row2d.at[bb], ir)
                pltpu.sync_copy(col2d.at[bb], ic)
                pltpu.async_copy(pp.at[ic], ab, s1)
                pltpu.async_copy(pm.at[ir], bbf, s2)

        for q in range(2):
            bb = w + (2 * i + q) * NW
            ir, ic, ab, bbf, s1, s2 = slots[q]

            @pl.when(bb < NBLK)
            def _(bb=bb, ab=ab, bbf=bbf, s1=s1, s2=s2):
                d3 = pltpu.async_copy(sarr.at[pl.ds(bb * K, K)], cbuf, sem3)
                pltpu.make_async_copy(pp, ab, s1).wait()
                pltpu.make_async_copy(pm, bbf, s2).wait()
                d3.wait()

                def compute(e, cy):
                    for j in range(8):
                        sl = pl.ds(j * 16, 16)
                        cv = cbuf[e, sl]
                        tpbuf[e, sl] = jnp.maximum(cv + ab[e, sl], 0.0)
                        tmbuf[e, sl] = jnp.maximum(cv + bbf[e, sl], 0.0)
                    return cy

                lax.fori_loop(0, K, compute, 0)
                pltpu.sync_copy(tpbuf, t2p_o.at[pl.ds(bb * K, K)])
                pltpu.sync_copy(tmbuf, t2m_o.at[pl.ds(bb * K, K)])

        return carry

    lax.fori_loop(0, (BPT + 1) // 2, body, 0)


def _sc_scatter1(ep, row2d, col2d, sp16, z64, z16,
                 ebp_o, ebm_o, ss_o,
                 up_sh, um_sh, ss_sh,
                 idx_r, idx_c, ebuf, p16, sem1, sem2):
    cc = lax.axis_index("c")
    s = lax.axis_index("s")
    w = s * NC + cc
    rows = pl.ds(s * ROWS_PT, ROWS_PT)

    pltpu.sync_copy(z64, up_sh.at[rows])
    pltpu.sync_copy(z64, um_sh.at[rows])
    pltpu.sync_copy(z16, ss_sh.at[rows])
    plsc.subcore_barrier()

    def body(i, carry):
        bb = w + i * NW

        @pl.when(bb < NBLK)
        def _():
            pltpu.sync_copy(row2d.at[bb], idx_r)
            pltpu.sync_copy(col2d.at[bb], idx_c)
            d1 = pltpu.async_copy(ep.at[pl.ds(bb * K, K)], ebuf, sem1)
            d2 = pltpu.async_copy(sp16.at[pl.ds(bb * K, K)], p16, sem2)
            d1.wait()
            d2.wait()
            pltpu.sync_copy(ebuf, up_sh.at[idx_r], add=True)
            pltpu.sync_copy(ebuf, um_sh.at[idx_c], add=True)
            pltpu.sync_copy(p16, ss_sh.at[idx_r], add=True)
            pltpu.sync_copy(p16, ss_sh.at[idx_c], add=True)

        return carry

    lax.fori_loop(0, BPT, body, 0)
    plsc.subcore_barrier()
    dst = pl.ds(cc * N + s * ROWS_PT, ROWS_PT)
    pltpu.sync_copy(up_sh.at[rows], ebp_o.at[dst])
    pltpu.sync_copy(um_sh.at[rows], ebm_o.at[dst])
    pltpu.sync_copy(ss_sh.at[rows], ss_o.at[dst])


def _sc_scatter2(epp, epm, row2d, col2d, z64,
                 vp_o, vm_o,
                 vp_sh, vm_sh,
                 idx_r, idx_c, pbuf, mbuf, sem1, sem2):
    cc = lax.axis_index("c")
    s = lax.axis_index("s")
    w = s * NC + cc
    rows = pl.ds(s * ROWS_PT, ROWS_PT)

    pltpu.sync_copy(z64, vp_sh.at[rows])
    pltpu.sync_copy(z64, vm_sh.at[rows])
    plsc.subcore_barrier()

    def body(i, carry):
        bb = w + i * NW

        @pl.when(bb < NBLK)
        def _():
            pltpu.sync_copy(row2d.at[bb], idx_r)
            pltpu.sync_copy(col2d.at[bb], idx_c)
            d1 = pltpu.async_copy(epp.at[pl.ds(bb * K, K)], pbuf, sem1)
            d2 = pltpu.async_copy(epm.at[pl.ds(bb * K, K)], mbuf, sem2)
            d1.wait()
            d2.wait()
            pltpu.sync_copy(pbuf, vp_sh.at[idx_c], add=True)
            pltpu.sync_copy(mbuf, vm_sh.at[idx_r], add=True)

        return carry

    lax.fori_loop(0, BPT, body, 0)
    plsc.subcore_barrier()
    dst = pl.ds(cc * N + s * ROWS_PT, ROWS_PT)
    pltpu.sync_copy(vp_sh.at[rows], vp_o.at[dst])
    pltpu.sync_copy(vm_sh.at[rows], vm_o.at[dst])


def _sds(shape, dtype=_F32):
    return jax.ShapeDtypeStruct(shape, dtype)


_gather1 = pl.kernel(
    _sc_gather1, mesh=_MESH, compiler_params=_SC_PARAMS,
    out_type=[_sds((E, 128))],
    scratch_types=[
        pltpu.VMEM((K,), jnp.int32), pltpu.VMEM((K,), jnp.int32),
        pltpu.VMEM((K, 128), _F32), pltpu.VMEM((K, 128), _F32),
        pltpu.VMEM((K, 128), _F32),
        pltpu.VMEM((K,), jnp.int32), pltpu.VMEM((K,), jnp.int32),
        pltpu.VMEM((K, 128), _F32), pltpu.VMEM((K, 128), _F32),
        pltpu.VMEM((K, 128), _F32),
        pltpu.VMEM((K, 128), _F32),
        pltpu.SemaphoreType.DMA, pltpu.SemaphoreType.DMA,
        pltpu.SemaphoreType.DMA, pltpu.SemaphoreType.DMA,
        pltpu.SemaphoreType.DMA, pltpu.SemaphoreType.DMA,
    ])

_gather2 = pl.kernel(
    _sc_gather2, mesh=_MESH, compiler_params=_SC_PARAMS,
    out_type=[_sds((E, 128)), _sds((E, 128))],
    scratch_types=[
        pltpu.VMEM((K,), jnp.int32), pltpu.VMEM((K,), jnp.int32),
        pltpu.VMEM((K, 128), _F32), pltpu.VMEM((K, 128), _F32),
        pltpu.VMEM((K,), jnp.int32), pltpu.VMEM((K,), jnp.int32),
        pltpu.VMEM((K, 128), _F32), pltpu.VMEM((K, 128), _F32),
        pltpu.VMEM((K, 128), _F32),
        pltpu.VMEM((K, 128), _F32), pltpu.VMEM((K, 128), _F32),
        pltpu.SemaphoreType.DMA, pltpu.SemaphoreType.DMA,
        pltpu.SemaphoreType.DMA, pltpu.SemaphoreType.DMA,
        pltpu.SemaphoreType.DMA,
    ])

_scatter1 = pl.kernel(
    _sc_scatter1, mesh=_MESH, compiler_params=_SC_PARAMS,
    out_type=[_sds((NC * N, 64)), _sds((NC * N, 64)), _sds((NC * N, 16))],
    scratch_types=[
        pltpu.VMEM_SHARED((N, 64), _F32), pltpu.VMEM_SHARED((N, 64), _F32),
        pltpu.VMEM_SHARED((N, 16), _F32),
        pltpu.VMEM((K,), jnp.int32), pltpu.VMEM((K,), jnp.int32),
        pltpu.VMEM((K, 64), _F32), pltpu.VMEM((K, 16), _F32),
        pltpu.SemaphoreType.DMA, pltpu.SemaphoreType.DMA,
    ])

_scatter2 = pl.kernel(
    _sc_scatter2, mesh=_MESH, compiler_params=_SC_PARAMS,
    out_type=[_sds((NC * N, 64)), _sds((NC * N, 64))],
    scratch_types=[
        pltpu.VMEM_SHARED((N, 64), _F32), pltpu.VMEM_SHARED((N, 64), _F32),
        pltpu.VMEM((K,), jnp.int32), pltpu.VMEM((K,), jnp.int32),
        pltpu.VMEM((K, 64), _F32), pltpu.VMEM((K, 64), _F32),
        pltpu.SemaphoreType.DMA, pltpu.SemaphoreType.DMA,
    ])


def _impl(x, edge_index, batch, edge_attr, glob_attr, p):
    r2 = lambda a: a.reshape(1, -1)
    w1 = p['phi_e1_W1']

    bn1, nbk = 2000, N // 2000
    ne, amat, bmat = pl.pallas_call(
        _tc_node_pre,
        grid=(nbk,),
        in_specs=[_rows(bn1, 2), _full((1, 64)), _full((1, 64)),
                  _full((128, 128)), _full((128, 128))],
        out_specs=[_rows(bn1, 128), _rows(bn1, 128), _rows(bn1, 128)],
        out_shape=[_sds((N, 128)), _sds((N, 128)), _sds((N, 128))],
    )(x, r2(p['enc1_node_W']), r2(p['enc1_node_b']),
      w1[0:128], w1[128:256])

    be, ebk = 2000, E // 2000
    cmat, smat, sp16 = pl.pallas_call(
        _tc_edge_pre,
        grid=(ebk,),
        in_specs=[_rows(be, 2), _full((1, 32)), _full((1, 32)),
                  _full((64, 128)), _full((1, 128)),
                  _full((1, 32)), _full((1, 32)),
                  _full((64, 128)), _full((1, 128))],
        out_specs=[_rows(be, 128), _rows(be, 128), _rows(be, 16)],
        out_shape=[_sds((E, 128)), _sds((E, 128)), _sds((E, 16))],
    )(edge_attr, r2(p['enc2_edge_W']), r2(p['enc2_edge_b']),
      w1[256:320], r2(p['phi_e1_b1']),
      r2(p['enc1_edge_W']), r2(p['enc1_edge_b']),
      p['phi_e2_W1'][0:64], r2(p['phi_e2_b1']))

    row2d = edge_index[0].reshape(NBLK, K)
    col2d = edge_index[1].reshape(NBLK, K)
    z64 = jnp.zeros((ROWS_PT, 64), _F32)
    z16 = jnp.zeros((ROWS_PT, 16), _F32)

    def w2call(t, w2, b2):
        return pl.pallas_call(
            _tc_w2, grid=(ebk,),
            in_specs=[_rows(be, 128), _full((128, 64)), _full((1, 64))],
            out_specs=_rows(be, 64),
            out_shape=_sds((E, 64)),
        )(t, w2, r2(b2))

    t1, = _gather1(amat, bmat, cmat, row2d, col2d)
    ep1 = w2call(t1, p['phi_e1_W2'], p['phi_e1_b2'])
    ebp_p, ebm_p, ss_p = _scatter1(ep1, row2d, col2d, sp16, z64, z16)

    bm, mbk = 2000, N // 2000
    w1v, pp, pm, idsp = pl.pallas_call(
        _tc_mid,
        grid=(mbk,),
        in_specs=[_parts(bm, 64)] * 2 + [_parts(bm, 16)] +
                 [_full((1, 32)), _full((1, 32)), _full((128, 128)),
                  _full((192, 128)), _full((1, 128)),
                  _full((128, 128)), _full((1, 128))],
        out_specs=[_rows(bm, 128), _rows(bm, 128), _rows(bm, 128),
                   _rows(bm, 64)],
        out_shape=[_sds((N, 128))] * 3 + [_sds((N, 64))],
    )(ebp_p.reshape(NC, N, 64), ebm_p.reshape(NC, N, 64),
      ss_p.reshape(NC, N, 16),
      r2(p['enc2_edge_W']), r2(p['enc2_edge_b']), p['phi_e2_W1'],
      p['phi_v1_W1'], r2(p['phi_v1_b1']),
      p['phi_v1_W2'], r2(p['phi_v1_b2']))

    t2p, t2m = _gather2(pp, pm, smat, row2d, col2d)
    ep2p = w2call(t2p, p['phi_e2_W2'], p['phi_e2_b2'])
    ep2m = w2call(t2m, p['phi_e2_W2'], p['phi_e2_b2'])
    vp_p, vm_p = _scatter2(ep2p, ep2m, row2d, col2d, z64)

    out = pl.pallas_call(
        _tc_final,
        grid=(mbk,),
        in_specs=[_rows(bm, 128), _rows(bm, 128)] +
                 [_parts(bm, 64)] * 2 +
                 [_rows(bm, 64),
                  pl.BlockSpec((1, 1, bm), lambda i: (i, 0, 0)),
                  _full((NB, 1)),
                  _full((1, 16)), _full((1, 16)),
                  _full((192, 128)), _full((1, 128)),
                  _full((128, 128)), _full((1, 128)),
                  _full((416, 128)), _full((1, 128)),
                  _full((128, 128)), _full((1, 128)),
                  _full((128, 2)), _full((1, 2))],
        out_specs=_rows(bm, 2),
        out_shape=_sds((N, 2)),
    )(ne, w1v,
      vp_p.reshape(NC, N, 64), vm_p.reshape(NC, N, 64),
      idsp, batch.reshape(mbk, 1, bm), glob_attr,
      r2(p['enc1_glob_W']), r2(p['enc1_glob_b']),
      p['phi_v2_W1'], r2(p['phi_v2_b1']),
      p['phi_v2_W2'], r2(p['phi_v2_b2']),
      p['ext_dec_W1'], r2(p['ext_dec_b1']),
      p['ext_dec_W2'], r2(p['ext_dec_b2']),
      p['dec_W'], r2(p['dec_b']))
    return out


_run = jax.jit(_impl)


def kernel(x, edge_index, batch, node_attr, edge_attr, glob_attr, params):
    del node_attr
    return _run(x, edge_index, batch, edge_attr, glob_attr, params)
